# Initial kernel scaffold; baseline (speedup 1.0000x reference)
import jax
import jax.numpy as jnp
from jax.experimental import pallas as pl


def kernel(down_proj_TED, weights_TX, indices_TX):
    raise NotImplementedError("write your pallas kernel here")


# speedup vs baseline: 2.1667x; 2.1667x over previous
import functools

import jax
import jax.numpy as jnp
from jax.experimental import pallas as pl

T, E, D, X = 4096, 8, 2048, 8
BT = 128


def _combine_body(dp_ref, w_ref, idx_ref, out_ref):
    w = w_ref[...]
    idx = idx_ref[...]
    dp = dp_ref[...]
    acc = None
    for e in range(E):
        we = jnp.sum(w * (idx == e).astype(jnp.float32), axis=1)
        term = dp[:, e, :] * we[:, None]
        acc = term if acc is None else acc + term
    out_ref[...] = acc


@jax.jit
def kernel(down_proj_TED, weights_TX, indices_TX):
    grid = (T // BT,)
    return pl.pallas_call(
        _combine_body,
        grid=grid,
        in_specs=[
            pl.BlockSpec((BT, E, D), lambda i: (i, 0, 0)),
            pl.BlockSpec((BT, X), lambda i: (i, 0)),
            pl.BlockSpec((BT, X), lambda i: (i, 0)),
        ],
        out_specs=pl.BlockSpec((BT, D), lambda i: (i, 0)),
        out_shape=jax.ShapeDtypeStruct((T, D), jnp.float32),
    )(down_proj_TED, weights_TX, indices_TX.astype(jnp.int32))


# speedup vs baseline: 2.4236x; 1.1186x over previous
import jax
import jax.numpy as jnp
from jax.experimental import pallas as pl

T, E, D, X = 4096, 8, 2048, 8
BT = 16


def _combine_body(dp_ref, w_ref, idx_ref, out_ref):
    w = w_ref[...]
    idx = idx_ref[...]
    dp = dp_ref[...]
    acc = None
    for e in range(E):
        we = jnp.sum(w * (idx == e).astype(jnp.float32), axis=1)
        term = dp[:, e * D:(e + 1) * D] * we[:, None]
        acc = term if acc is None else acc + term
    out_ref[...] = acc


@jax.jit
def kernel(down_proj_TED, weights_TX, indices_TX):
    dp2 = down_proj_TED.reshape(T, E * D)
    grid = (T // BT,)
    return pl.pallas_call(
        _combine_body,
        grid=grid,
        in_specs=[
            pl.BlockSpec((BT, E * D), lambda i: (i, 0)),
            pl.BlockSpec((BT, X), lambda i: (i, 0)),
            pl.BlockSpec((BT, X), lambda i: (i, 0)),
        ],
        out_specs=pl.BlockSpec((BT, D), lambda i: (i, 0)),
        out_shape=jax.ShapeDtypeStruct((T, D), jnp.float32),
    )(dp2, weights_TX, indices_TX.astype(jnp.int32))


# speedup vs baseline: 7.3953x; 3.0513x over previous
import jax
import jax.numpy as jnp
from jax.experimental import pallas as pl

T, E, D, X = 4096, 8, 2048, 8
BT = 128


def _combine_body(dp_ref, w_ref, idx_ref, out_ref):
    w = w_ref[...]
    idx = idx_ref[...]
    dp = dp_ref[...]
    e_row = jax.lax.broadcasted_iota(jnp.int32, (BT * E, X), 0) % E
    wrow = jnp.sum(w * (idx == e_row).astype(jnp.float32), axis=1,
                   keepdims=True)
    prod = dp * wrow
    out_ref[...] = prod.reshape(BT, E, D).sum(axis=1)


@jax.jit
def kernel(down_proj_TED, weights_TX, indices_TX):
    dp2 = down_proj_TED.reshape(T * E, D)
    w_exp = jnp.repeat(weights_TX, E, axis=0)
    idx_exp = jnp.repeat(indices_TX.astype(jnp.int32), E, axis=0)
    grid = (T // BT,)
    return pl.pallas_call(
        _combine_body,
        grid=grid,
        in_specs=[
            pl.BlockSpec((BT * E, D), lambda i: (i, 0)),
            pl.BlockSpec((BT * E, X), lambda i: (i, 0)),
            pl.BlockSpec((BT * E, X), lambda i: (i, 0)),
        ],
        out_specs=pl.BlockSpec((BT, D), lambda i: (i, 0)),
        out_shape=jax.ShapeDtypeStruct((T, D), jnp.float32),
    )(dp2, w_exp, idx_exp)
